# baseline (speedup 1.0000x reference)
import jax
import jax.numpy as jnp
from jax import lax
from jax.experimental import pallas as pl
from jax.experimental.pallas import tpu as pltpu
from jax.experimental.pallas import tpu_sc as plsc

BATCH = 16384
INP_DIM = 39
N_CONT = 13
N_CAT = 26
VOCAB = 100000
HALF_V = VOCAB // 2
EMB_DIM = 32
EMB_TOT = N_CAT * EMB_DIM
OUT_DIM = N_CONT + EMB_TOT

NC, NS, L = 2, 16, 16
NW = NC * NS
ROWS_PER_W = EMB_TOT // NW
BCH = 2048
NB = BATCH // BCH
UNROLL = 8
KC = BCH // (L * UNROLL)


def _sc_body(xt_hbm, t2f_hbm, out_hbm, rowb, idxb, outb, rsem, isem, osem):
    wid = lax.axis_index("s") * NC + lax.axis_index("c")
    r0 = wid * ROWS_PER_W

    def row_half(r, h):
        return pltpu.make_async_copy(
            t2f_hbm.at[pl.ds(r * VOCAB + h * HALF_V, HALF_V)],
            rowb.at[pl.ds(h * HALF_V, HALF_V)],
            rsem.at[h],
        )

    def idx_chunk(xrow, j):
        return pltpu.make_async_copy(
            xt_hbm.at[xrow, pl.ds(j * BCH, BCH)],
            idxb.at[pl.ds((j % 2) * BCH, BCH)],
            isem.at[j % 2],
        )

    def out_chunk(row, j):
        return pltpu.make_async_copy(
            outb.at[pl.ds(j * BCH, BCH)],
            out_hbm.at[row, pl.ds(j * BCH, BCH)],
            osem.at[j],
        )

    @pl.when(wid < N_CONT)
    def _cont():
        idx_chunk(wid, 0).start()
        for j in range(NB):
            if j + 1 < NB:
                idx_chunk(wid, j + 1).start()
            idx_chunk(wid, j).wait()

            def conv_body(k, _):
                for u in range(UNROLL):
                    o = (k * UNROLL + u) * L
                    v = idxb[pl.ds((j % 2) * BCH + o, L)]
                    outb[pl.ds(j * BCH + o, L)] = v.astype(jnp.float32)
                return _
            lax.fori_loop(0, KC, conv_body, None)
            out_chunk(wid, j).start()
        for j in range(NB):
            out_chunk(wid, j).wait()

    row_half(r0, 0).start()
    row_half(r0, 1).start()

    def row_body(i, _):
        r = r0 + i
        c = lax.div(r, EMB_DIM)

        row_half(r, 0).wait()
        idx_chunk(N_CONT + c, 0).start()
        for j in range(NB):
            if j + 1 < NB:
                idx_chunk(N_CONT + c, j + 1).start()
            idx_chunk(N_CONT + c, j).wait()

            @pl.when(i > 0)
            def _drain():
                out_chunk(N_CONT + r, j).wait()

            def p0_body(k, _):
                for u in range(UNROLL):
                    o = (k * UNROLL + u) * L
                    iv = idxb[pl.ds((j % 2) * BCH + o, L)]
                    m = iv < HALF_V
                    v = plsc.load_gather(rowb, [jnp.where(m, iv, 0)])
                    outb[pl.ds(j * BCH + o, L)] = jnp.where(m, v, 0.0)
                return _
            lax.fori_loop(0, KC, p0_body, None)

        @pl.when(i + 1 < ROWS_PER_W)
        def _pf0():
            row_half(r + 1, 0).start()

        row_half(r, 1).wait()
        idx_chunk(N_CONT + c, 0).start()
        for j in range(NB):
            if j + 1 < NB:
                idx_chunk(N_CONT + c, j + 1).start()
            idx_chunk(N_CONT + c, j).wait()

            def p1_body(k, _):
                for u in range(UNROLL):
                    o = (k * UNROLL + u) * L
                    iv = idxb[pl.ds((j % 2) * BCH + o, L)]
                    m = iv >= HALF_V
                    v = plsc.load_gather(rowb, [iv])
                    prev = outb[pl.ds(j * BCH + o, L)]
                    outb[pl.ds(j * BCH + o, L)] = jnp.where(m, v, prev)
                return _
            lax.fori_loop(0, KC, p1_body, None)
            out_chunk(N_CONT + r, j).start()

        @pl.when(i + 1 < ROWS_PER_W)
        def _pf1():
            row_half(r + 1, 1).start()
        return _

    lax.fori_loop(0, ROWS_PER_W, row_body, None)

    for j in range(NB):
        out_chunk(N_CONT + r0 + ROWS_PER_W - 1, j).wait()


@jax.jit
def _run(xt, t2f):
    mesh = plsc.VectorSubcoreMesh(
        core_axis_name="c", subcore_axis_name="s", num_cores=NC, num_subcores=NS
    )
    out_t = pl.kernel(
        _sc_body,
        out_type=jax.ShapeDtypeStruct((OUT_DIM, BATCH), jnp.float32),
        mesh=mesh,
        compiler_params=pltpu.CompilerParams(needs_layout_passes=False),
        scratch_types=[
            pltpu.VMEM((VOCAB,), jnp.float32),
            pltpu.VMEM((2 * BCH,), jnp.int32),
            pltpu.VMEM((BATCH,), jnp.float32),
            pltpu.SemaphoreType.DMA((2,)),
            pltpu.SemaphoreType.DMA((2,)),
            pltpu.SemaphoreType.DMA((NB,)),
        ],
    )(xt, t2f)
    return out_t.T


def kernel(x, tables):
    xt = x.astype(jnp.int32).T
    t2f = tables.transpose(0, 2, 1).reshape(-1)
    return _run(xt, t2f)


# speedup vs baseline: 3.3239x; 3.3239x over previous
import jax
import jax.numpy as jnp
from jax import lax
from jax.experimental import pallas as pl
from jax.experimental.pallas import tpu as pltpu
from jax.experimental.pallas import tpu_sc as plsc

BATCH = 16384
INP_DIM = 39
N_CONT = 13
N_CAT = 26
VOCAB = 100000
EMB_DIM = 32
EMB_TOT = N_CAT * EMB_DIM
OUT_DIM = N_CONT + EMB_TOT

NC, NS, L = 2, 16, 16
NW = NC * NS
ROWS_PER_W = EMB_TOT // NW
QTR = BATCH // 4
NQ = 4
UNROLL = 16
KQ = QTR // (L * UNROLL)


def _sc_body(xt_hbm, t2_hbm, out_hbm, rowb, idxb, outb, osem):
    wid = lax.axis_index("s") * NC + lax.axis_index("c")

    def out_qtr(row, q):
        return pltpu.make_async_copy(
            outb.at[pl.ds((q % 2) * QTR, QTR)],
            out_hbm.at[row, pl.ds(q * QTR, QTR)],
            osem.at[q % 2],
        )

    @pl.when(wid < N_CONT)
    def _cont():
        pltpu.sync_copy(xt_hbm.at[wid, :], idxb)
        for q in range(NQ):
            def conv_body(k, _):
                for u in range(UNROLL):
                    o = k * L * UNROLL + u * L
                    v = idxb[pl.ds(q * QTR + o, L)]
                    outb[pl.ds((q % 2) * QTR + o, L)] = v.astype(jnp.float32)
                return _
            lax.fori_loop(0, KQ, conv_body, None)
            pltpu.sync_copy(
                outb.at[pl.ds((q % 2) * QTR, QTR)],
                out_hbm.at[wid, pl.ds(q * QTR, QTR)],
            )

    def row_body(i, c_prev):
        r = wid * ROWS_PER_W + i
        c = lax.div(r, EMB_DIM)

        @pl.when(c != c_prev)
        def _load_idx():
            pltpu.sync_copy(xt_hbm.at[N_CONT + c, :], idxb)

        pltpu.sync_copy(t2_hbm.at[r, :], rowb)
        for q in range(NQ):
            @pl.when(jnp.logical_or(i > 0, q >= 2))
            def _drain():
                out_qtr(N_CONT + r, q).wait()

            def gat_body(k, _):
                for u in range(UNROLL):
                    o = k * L * UNROLL + u * L
                    iv = idxb[pl.ds(q * QTR + o, L)]
                    outb[pl.ds((q % 2) * QTR + o, L)] = plsc.load_gather(
                        rowb, [iv]
                    )
                return _
            lax.fori_loop(0, KQ, gat_body, None)
            out_qtr(N_CONT + r, q).start()
        return c

    lax.fori_loop(0, ROWS_PER_W, row_body, jnp.int32(-1))

    last = wid * ROWS_PER_W + ROWS_PER_W - 1
    for q in range(2):
        out_qtr(N_CONT + last, q).wait()


@jax.jit
def _run(xt, t2):
    mesh = plsc.VectorSubcoreMesh(
        core_axis_name="c", subcore_axis_name="s", num_cores=NC, num_subcores=NS
    )
    out_t = pl.kernel(
        _sc_body,
        out_type=jax.ShapeDtypeStruct((OUT_DIM, BATCH), jnp.float32),
        mesh=mesh,
        compiler_params=pltpu.CompilerParams(needs_layout_passes=False),
        scratch_types=[
            pltpu.VMEM((VOCAB,), jnp.float32),
            pltpu.VMEM((BATCH,), jnp.int32),
            pltpu.VMEM((2 * QTR,), jnp.float32),
            pltpu.SemaphoreType.DMA((2,)),
        ],
    )(xt, t2)
    return out_t.T


def kernel(x, tables):
    xt = x.astype(jnp.int32).T
    t2 = tables.transpose(0, 2, 1).reshape(EMB_TOT, VOCAB)
    return _run(xt, t2)
